# Initial kernel scaffold; baseline (speedup 1.0000x reference)
import jax
import jax.numpy as jnp
from jax.experimental import pallas as pl


def kernel(numeric, cat_idx, mean, var):
    raise NotImplementedError("write your pallas kernel here")


# speedup vs baseline: 1.9190x; 1.9190x over previous
import jax
import jax.numpy as jnp
from jax.experimental import pallas as pl

BATCH = 1024
NUM_NUMERIC = 13
NUM_CAT = 26
DEPTH = 1001
OUT_COLS = NUM_CAT * DEPTH + NUM_NUMERIC

BLOCK = 128


def _body(num_ref, cat_ref, mean_ref, var_ref, out_ref):
    iota = jax.lax.broadcasted_iota(jnp.int32, (BLOCK, DEPTH), 1)
    for f in range(NUM_CAT):
        sel = cat_ref[:, f : f + 1]
        out_ref[:, f * DEPTH : (f + 1) * DEPTH] = (iota == sel).astype(jnp.float32)
    inv = 1.0 / jnp.maximum(jnp.sqrt(var_ref[...]), 1e-7)
    out_ref[:, NUM_CAT * DEPTH :] = (num_ref[...] - mean_ref[...]) * inv


def kernel(numeric, cat_idx, mean, var):
    grid = (BATCH // BLOCK,)
    return pl.pallas_call(
        _body,
        grid=grid,
        in_specs=[
            pl.BlockSpec((BLOCK, NUM_NUMERIC), lambda i: (i, 0)),
            pl.BlockSpec((BLOCK, NUM_CAT), lambda i: (i, 0)),
            pl.BlockSpec((1, NUM_NUMERIC), lambda i: (0, 0)),
            pl.BlockSpec((1, NUM_NUMERIC), lambda i: (0, 0)),
        ],
        out_specs=pl.BlockSpec((BLOCK, OUT_COLS), lambda i: (i, 0)),
        out_shape=jax.ShapeDtypeStruct((BATCH, OUT_COLS), jnp.float32),
    )(numeric, cat_idx, mean.reshape(1, -1), var.reshape(1, -1))
